# baseline (speedup 1.0000x reference)
import jax
import jax.numpy as jnp
from jax.experimental import pallas as pl
from jax.experimental.pallas import tpu as pltpu

_ROWS_PER_BLOCK = 128


def _surface_kernel(x_ref, o_ref):
    x = x_ref[...]
    R, D = x.shape
    m = jnp.max(x, axis=-1, keepdims=True)
    e1 = jnp.exp(x - m)
    s1 = jnp.sum(e1, axis=-1, keepdims=True)
    a = e1 * (1.0 / s1)
    mn = jnp.exp(jnp.min(x, axis=-1, keepdims=True) - m)
    e2 = jnp.exp(mn - e1)
    s2 = jnp.sum(e2, axis=-1, keepdims=True)
    b = e2 * (1.0 / s2)

    row = jax.lax.broadcasted_iota(jnp.int32, (8, D), 0)
    idx_lo = row >> 1
    idx_hi = idx_lo + 4
    odd = (row & 1) == 1
    for t in range(R // 8):
        at = a[8 * t : 8 * (t + 1)]
        bt = b[8 * t : 8 * (t + 1)]
        o_ref[16 * t : 16 * t + 8, :] = jnp.where(
            odd,
            jnp.take_along_axis(bt, idx_lo, axis=0),
            jnp.take_along_axis(at, idx_lo, axis=0),
        )
        o_ref[16 * t + 8 : 16 * t + 16, :] = jnp.where(
            odd,
            jnp.take_along_axis(bt, idx_hi, axis=0),
            jnp.take_along_axis(at, idx_hi, axis=0),
        )


def kernel(batch):
    B, J, D = batch.shape
    N = B * J
    x2 = batch.reshape(N, D)
    R = _ROWS_PER_BLOCK
    grid = (N // R,)
    out = pl.pallas_call(
        _surface_kernel,
        grid=grid,
        in_specs=[pl.BlockSpec((R, D), lambda i: (i, 0))],
        out_specs=pl.BlockSpec((2 * R, D), lambda i: (i, 0)),
        out_shape=jax.ShapeDtypeStruct((2 * N, D), batch.dtype),
        compiler_params=pltpu.CompilerParams(
            dimension_semantics=("parallel",),
        ),
    )(x2)
    return out.reshape(B, 2 * J, D)


# speedup vs baseline: 1.1149x; 1.1149x over previous
import jax
import jax.numpy as jnp
from jax.experimental import pallas as pl
from jax.experimental.pallas import tpu as pltpu

_ROWS_PER_BLOCK = 256


def _surface_kernel(x_ref, o_ref):
    x = x_ref[...]
    R, D = x.shape
    m = jnp.max(x, axis=-1, keepdims=True)
    e1 = jnp.exp(x - m)
    s1 = jnp.sum(e1, axis=-1, keepdims=True)
    a = e1 * (1.0 / s1)
    mn = jnp.exp(jnp.min(x, axis=-1, keepdims=True) - m)
    e2 = jnp.exp(mn - e1)
    s2 = jnp.sum(e2, axis=-1, keepdims=True)
    b = e2 * (1.0 / s2)

    row = jax.lax.broadcasted_iota(jnp.int32, (8, D), 0)
    idx_lo = row >> 1
    idx_hi = idx_lo + 4
    odd = (row & 1) == 1
    for t in range(R // 8):
        at = a[8 * t : 8 * (t + 1)]
        bt = b[8 * t : 8 * (t + 1)]
        o_ref[16 * t : 16 * t + 8, :] = jnp.where(
            odd,
            jnp.take_along_axis(bt, idx_lo, axis=0),
            jnp.take_along_axis(at, idx_lo, axis=0),
        )
        o_ref[16 * t + 8 : 16 * t + 16, :] = jnp.where(
            odd,
            jnp.take_along_axis(bt, idx_hi, axis=0),
            jnp.take_along_axis(at, idx_hi, axis=0),
        )


def kernel(batch):
    B, J, D = batch.shape
    N = B * J
    x2 = batch.reshape(N, D)
    R = _ROWS_PER_BLOCK
    grid = (N // R,)
    out = pl.pallas_call(
        _surface_kernel,
        grid=grid,
        in_specs=[pl.BlockSpec((R, D), lambda i: (i, 0))],
        out_specs=pl.BlockSpec((2 * R, D), lambda i: (i, 0)),
        out_shape=jax.ShapeDtypeStruct((2 * N, D), batch.dtype),
        compiler_params=pltpu.CompilerParams(
            dimension_semantics=("parallel",),
        ),
    )(x2)
    return out.reshape(B, 2 * J, D)


# speedup vs baseline: 1.1238x; 1.0080x over previous
import jax
import jax.numpy as jnp
from jax.experimental import pallas as pl
from jax.experimental.pallas import tpu as pltpu

_ROWS_PER_BLOCK = 256


def _surface_kernel(x_ref, o_ref):
    R, D = x_ref.shape
    row = jax.lax.broadcasted_iota(jnp.int32, (8, D), 0)
    idx_lo = row >> 1
    idx_hi = idx_lo + 4
    odd = (row & 1) == 1
    for t in range(R // 8):
        xt = x_ref[8 * t : 8 * (t + 1), :]
        m = jnp.max(xt, axis=-1, keepdims=True)
        e1 = jnp.exp(xt - m)
        s1 = jnp.sum(e1, axis=-1, keepdims=True)
        a = e1 * (1.0 / s1)
        mn = jnp.exp(jnp.min(xt, axis=-1, keepdims=True) - m)
        e2 = jnp.exp(mn - e1)
        s2 = jnp.sum(e2, axis=-1, keepdims=True)
        b = e2 * (1.0 / s2)
        o_ref[16 * t : 16 * t + 8, :] = jnp.where(
            odd,
            jnp.take_along_axis(b, idx_lo, axis=0),
            jnp.take_along_axis(a, idx_lo, axis=0),
        )
        o_ref[16 * t + 8 : 16 * t + 16, :] = jnp.where(
            odd,
            jnp.take_along_axis(b, idx_hi, axis=0),
            jnp.take_along_axis(a, idx_hi, axis=0),
        )


def kernel(batch):
    B, J, D = batch.shape
    N = B * J
    x2 = batch.reshape(N, D)
    R = _ROWS_PER_BLOCK
    grid = (N // R,)
    out = pl.pallas_call(
        _surface_kernel,
        grid=grid,
        in_specs=[pl.BlockSpec((R, D), lambda i: (i, 0))],
        out_specs=pl.BlockSpec((2 * R, D), lambda i: (i, 0)),
        out_shape=jax.ShapeDtypeStruct((2 * N, D), batch.dtype),
        compiler_params=pltpu.CompilerParams(
            dimension_semantics=("parallel",),
        ),
    )(x2)
    return out.reshape(B, 2 * J, D)
